# Initial kernel scaffold; baseline (speedup 1.0000x reference)
import jax
import jax.numpy as jnp
from jax.experimental import pallas as pl


def kernel(inputs, targets):
    raise NotImplementedError("write your pallas kernel here")


# speedup vs baseline: 21.7040x; 21.7040x over previous
import jax
import jax.numpy as jnp
from jax.experimental import pallas as pl

_GAMMA = 0.1
_BIG = 1e10
_N = 4096


def _sdtw_kernel(x_ref, y0_ref, o_ref):
    n = _N
    x = x_ref[...]
    y0 = y0_ref[...]
    lane = jax.lax.broadcasted_iota(jnp.int32, (1, n), 1)
    big = jnp.float32(_BIG)
    bigvec = x * jnp.float32(0.0) + big
    inv_g = jnp.float32(1.0 / _GAMMA)
    g = jnp.float32(_GAMMA)

    def step(t, carry):
        r2s, r1, r1s, yr = carry
        j = t - lane
        valid = (j >= 0) & (j < n)
        d = (x - yr) ** 2
        up = jnp.where(lane > 0, r1s, big)
        left = jnp.where(j > 0, r1, big)
        diag = jnp.where((lane > 0) & (j > 0), r2s, big)
        diag = jnp.where((lane == 0) & (j == 0), jnp.float32(0.0), diag)
        m = jnp.minimum(jnp.minimum(up, left), diag)
        s = (jnp.exp((m - up) * inv_g)
             + jnp.exp((m - left) * inv_g)
             + jnp.exp((m - diag) * inv_g))
        smin = m - g * jnp.log(s)
        r = jnp.where(valid, d + smin, big)
        rs = jnp.roll(r, 1, axis=1)
        yr_next = jnp.roll(yr, 1, axis=1)
        return (r1s, r, rs, yr_next)

    init = (bigvec, bigvec, bigvec, y0)
    _, r_last, _, _ = jax.lax.fori_loop(0, 2 * n - 1, step, init)
    o_ref[...] = jnp.abs(r_last[:, n - 1:n])


def kernel(inputs, targets):
    n = _N
    y0 = jnp.roll(targets[::-1], 1 - n).reshape(1, n).astype(jnp.float32)
    out = pl.pallas_call(
        _sdtw_kernel,
        out_shape=jax.ShapeDtypeStruct((1, 1), jnp.float32),
    )(inputs.reshape(1, n).astype(jnp.float32), y0)
    return out[0, 0]


# speedup vs baseline: 39.4615x; 1.8182x over previous
import jax
import jax.numpy as jnp
from jax.experimental import pallas as pl

_GAMMA = 0.1
_BIG = 1e10
_N = 4096
_R = 8
_C = _N // _R


def _sdtw_kernel(x_ref, y0_ref, o_ref):
    n = _N
    x = x_ref[...]
    y0 = y0_ref[...]
    row = jax.lax.broadcasted_iota(jnp.int32, (_R, _C), 0)
    col = jax.lax.broadcasted_iota(jnp.int32, (_R, _C), 1)
    fi = row * _C + col
    col0 = col == 0
    row_h = jax.lax.broadcasted_iota(jnp.int32, (_R, 1), 0)
    big = jnp.float32(_BIG)
    bigvec = x * jnp.float32(0.0) + big
    inv_g = jnp.float32(1.0 / _GAMMA)
    g = jnp.float32(_GAMMA)

    def roll_flat(v):
        b = jnp.roll(v, 1, axis=1)
        c0 = jnp.roll(b[:, 0:1], 1, axis=0)
        return jnp.where(col0, c0, b)

    def roll_state(v):
        b = jnp.roll(v, 1, axis=1)
        c0 = jnp.roll(b[:, 0:1], 1, axis=0)
        c0 = jnp.where(row_h == 0, big, c0)
        return jnp.where(col0, c0, b)

    def step(t, carry):
        r2s, r1, r1s, yr = carry
        jpos = fi < t
        valid = (fi <= t) & (fi > t - n)
        d = (x - yr) ** 2
        up = r1s
        left = jnp.where(jpos, r1, big)
        diag = jnp.where(jpos, r2s, big)
        diag = jnp.where((t == 0) & (fi == 0), jnp.float32(0.0), diag)
        m = jnp.minimum(jnp.minimum(up, left), diag)
        s = (jnp.exp((m - up) * inv_g)
             + jnp.exp((m - left) * inv_g)
             + jnp.exp((m - diag) * inv_g))
        smin = m - g * jnp.log(s)
        r = jnp.where(valid, d + smin, big)
        return (r1s, r, roll_state(r), roll_flat(yr))

    init = (bigvec, bigvec, bigvec, y0)
    _, r_last, _, _ = jax.lax.fori_loop(0, 2 * n - 1, step, init)
    o_ref[...] = jnp.abs(r_last[_R - 1:_R, _C - 1:_C])


def kernel(inputs, targets):
    n = _N
    y0 = jnp.roll(targets[::-1], 1 - n).reshape(_R, _C).astype(jnp.float32)
    out = pl.pallas_call(
        _sdtw_kernel,
        out_shape=jax.ShapeDtypeStruct((1, 1), jnp.float32),
    )(inputs.reshape(_R, _C).astype(jnp.float32), y0)
    return out[0, 0]


# speedup vs baseline: 62.5865x; 1.5860x over previous
import jax
import jax.numpy as jnp
from jax.experimental import pallas as pl

_GAMMA = 0.1
_BIG = 1e10
_N = 4096
_R = 8
_C = _N // _R


def _sdtw_kernel(x_ref, y0_ref, o_ref):
    n = _N
    x = x_ref[...]
    y0 = y0_ref[...]
    row = jax.lax.broadcasted_iota(jnp.int32, (_R, _C), 0)
    col = jax.lax.broadcasted_iota(jnp.int32, (_R, _C), 1)
    fi = row * _C + col
    col0 = col == 0
    row_h = jax.lax.broadcasted_iota(jnp.int32, (_R, 1), 0)
    big = jnp.float32(_BIG)
    bigvec = x * jnp.float32(0.0) + big
    inv_g = jnp.float32(1.0 / _GAMMA)
    g = jnp.float32(_GAMMA)

    row0col0 = (row == 0) & col0

    def roll_flat(v):
        b = jnp.roll(v, 1, axis=1)
        sub = jnp.roll(b, 1, axis=0)
        return jnp.where(col0, sub, b)

    def roll_state(v):
        b = jnp.roll(v, 1, axis=1)
        sub = jnp.roll(b, 1, axis=0)
        return jnp.where(row0col0, big, jnp.where(col0, sub, b))

    def step(t, carry):
        r2s, r1, r1s, yr = carry
        jpos = fi < t
        valid = (fi <= t) & (fi > t - n)
        d = (x - yr) ** 2
        up = r1s
        left = jnp.where(jpos, r1, big)
        diag = jnp.where(jpos, r2s, big)
        diag = jnp.where((t == 0) & (fi == 0), jnp.float32(0.0), diag)
        m = jnp.minimum(jnp.minimum(up, left), diag)
        s = (jnp.exp((m - up) * inv_g)
             + jnp.exp((m - left) * inv_g)
             + jnp.exp((m - diag) * inv_g))
        smin = m - g * jnp.log(s)
        r = jnp.where(valid, d + smin, big)
        return (r1s, r, roll_state(r), roll_flat(yr))

    init = (bigvec, bigvec, bigvec, y0)
    _, r_last, _, _ = jax.lax.fori_loop(0, 2 * n - 1, step, init)
    o_ref[...] = jnp.abs(r_last[_R - 1:_R, _C - 1:_C])


def kernel(inputs, targets):
    n = _N
    y0 = jnp.roll(targets[::-1], 1 - n).reshape(_R, _C).astype(jnp.float32)
    out = pl.pallas_call(
        _sdtw_kernel,
        out_shape=jax.ShapeDtypeStruct((1, 1), jnp.float32),
    )(inputs.reshape(_R, _C).astype(jnp.float32), y0)
    return out[0, 0]


# speedup vs baseline: 62.8932x; 1.0049x over previous
import jax
import jax.numpy as jnp
from jax.experimental import pallas as pl

_GAMMA = 0.1
_BIG = 1e10
_N = 4096
_R = 8
_C = _N // _R
_U = 1
_T = 2 * _N - 1
_NBLK = _T // _U
_TAIL = _T - _NBLK * _U


def _sdtw_kernel(x_ref, y0_ref, o_ref):
    n = _N
    x = x_ref[...]
    y0 = y0_ref[...]
    row = jax.lax.broadcasted_iota(jnp.int32, (_R, _C), 0)
    col = jax.lax.broadcasted_iota(jnp.int32, (_R, _C), 1)
    fi = row * _C + col
    col0 = col == 0
    row0col0 = (row == 0) & col0
    big = jnp.float32(_BIG)
    bigvec = x * jnp.float32(0.0) + big
    inv_g2 = jnp.float32(1.4426950408889634 / _GAMMA)
    g_ln2 = jnp.float32(_GAMMA * 0.6931471805599453)

    def roll_flat_k(v, k):
        b = jnp.roll(v, k, axis=1)
        sub = jnp.roll(b, 1, axis=0)
        return jnp.where(col < k, sub, b)

    def roll_state(v):
        b = jnp.roll(v, 1, axis=1)
        sub = jnp.roll(b, 1, axis=0)
        return jnp.where(row0col0, big, jnp.where(col0, sub, b))

    def step(t, carry, yr):
        r2s, r1, r1s = carry
        jpos = fi < t
        valid = (fi <= t) & (fi > t - n)
        d = (x - yr) ** 2
        up = r1s
        left = jnp.where(jpos, r1, big)
        diag = jnp.where(jpos, r2s, big)
        diag = jnp.where((t == 0) & (fi == 0), jnp.float32(0.0), diag)
        m = jnp.minimum(jnp.minimum(up, left), diag)
        s = (jnp.exp2((m - up) * inv_g2)
             + jnp.exp2((m - left) * inv_g2)
             + jnp.exp2((m - diag) * inv_g2))
        smin = m - g_ln2 * jnp.log2(s)
        r = jnp.where(valid, d + smin, big)
        return (r1s, r, roll_state(r))

    def block(b, carry):
        st = (carry[0], carry[1], carry[2])
        yr = carry[3]
        t0 = b * _U
        for k in range(_U):
            yr_k = yr if k == 0 else roll_flat_k(yr, k)
            st = step(t0 + k, st, yr_k)
        return st + (roll_flat_k(yr, _U),)

    init = (bigvec, bigvec, bigvec, y0)
    carry = jax.lax.fori_loop(0, _NBLK, block, init)
    st = (carry[0], carry[1], carry[2])
    yr = carry[3]
    for k in range(_TAIL):
        yr_k = yr if k == 0 else roll_flat_k(yr, k)
        st = step(_NBLK * _U + k, st, yr_k)
    o_ref[...] = jnp.abs(st[1][_R - 1:_R, _C - 1:_C])


def kernel(inputs, targets):
    n = _N
    y0 = jnp.roll(targets[::-1], 1 - n).reshape(_R, _C).astype(jnp.float32)
    out = pl.pallas_call(
        _sdtw_kernel,
        out_shape=jax.ShapeDtypeStruct((1, 1), jnp.float32),
    )(inputs.reshape(_R, _C).astype(jnp.float32), y0)
    return out[0, 0]


# speedup vs baseline: 63.0257x; 1.0021x over previous
import jax
import jax.numpy as jnp
from jax.experimental import pallas as pl

_GAMMA = 0.1
_BIG = 1e10
_N = 4096
_R = 8
_C = _N // _R


def _sdtw_kernel(x_ref, y0_ref, o_ref):
    n = _N
    x = x_ref[...]
    y0 = y0_ref[...]
    row = jax.lax.broadcasted_iota(jnp.int32, (_R, _C), 0)
    col = jax.lax.broadcasted_iota(jnp.int32, (_R, _C), 1)
    fi = row * _C + col
    col0 = col == 0
    row0col0 = (row == 0) & col0
    big = jnp.float32(_BIG)
    bigvec = x * jnp.float32(0.0) + big
    inv_g2 = jnp.float32(1.4426950408889634 / _GAMMA)
    g_ln2 = jnp.float32(_GAMMA * 0.6931471805599453)

    def roll_flat(v):
        b = jnp.roll(v, 1, axis=1)
        sub = jnp.roll(b, 1, axis=0)
        return jnp.where(col0, sub, b)

    def roll_state(v):
        b = jnp.roll(v, 1, axis=1)
        sub = jnp.roll(b, 1, axis=0)
        return jnp.where(row0col0, big, jnp.where(col0, sub, b))

    def step(t, carry):
        r2s, r1, r1s, yr = carry
        jpos = fi < t
        valid = (fi <= t) & (fi > t - n)
        d = (x - yr) ** 2
        up = r1s
        left = jnp.where(jpos, r1, big)
        diag = jnp.where(jpos, r2s, big)
        diag = jnp.where((t == 0) & (fi == 0), jnp.float32(0.0), diag)
        m = jnp.minimum(jnp.minimum(up, left), diag)
        s = (jnp.exp2((m - up) * inv_g2)
             + jnp.exp2((m - left) * inv_g2)
             + jnp.exp2((m - diag) * inv_g2))
        smin = m - g_ln2 * jnp.log2(s)
        r = jnp.where(valid, d + smin, big)
        return (r1s, r, roll_state(r), roll_flat(yr))

    init = (bigvec, bigvec, bigvec, y0)
    _, r_last, _, _ = jax.lax.fori_loop(0, 2 * n - 1, step, init)
    o_ref[...] = jnp.abs(r_last[_R - 1:_R, _C - 1:_C])


def kernel(inputs, targets):
    n = _N
    y0 = jnp.roll(targets[::-1], 1 - n).reshape(_R, _C).astype(jnp.float32)
    out = pl.pallas_call(
        _sdtw_kernel,
        out_shape=jax.ShapeDtypeStruct((1, 1), jnp.float32),
    )(inputs.reshape(_R, _C).astype(jnp.float32), y0)
    return out[0, 0]
